# baseline (speedup 1.0000x reference)
import functools

import jax
import jax.numpy as jnp
from jax import lax
from jax.experimental import pallas as pl
from jax.experimental.pallas import tpu as pltpu
from jax.experimental.pallas import tpu_sc as plsc

N_NODES = 10000
N_EDGES = 320000
D_X = 120
IN_CH = 128
HID = 128
N_GRAPHS = 64

NC = 2
NS = 16
NW = NC * NS
CHUNK = 128
K = 80
E_PAD = NW * K * CHUNK
SW = 40
ACC_ROWS = 10112
RPS = ACC_ROWS // NS
DEG_ROWS = 10240
DRPS = DEG_ROWS // NS
RBLK = 2000
GRID = N_NODES // RBLK

_mesh = plsc.VectorSubcoreMesh(core_axis_name="c", subcore_axis_name="s")


def _sc_degree(dst2d, z128, o128):
    @functools.partial(
        pl.kernel, mesh=_mesh,
        out_type=jax.ShapeDtypeStruct((NC, DEG_ROWS, HID), jnp.float32),
        scratch_types=[
            pltpu.VMEM((K, CHUNK), jnp.int32),
            pltpu.VMEM((CHUNK, HID), jnp.float32),
            pltpu.VMEM_SHARED((DEG_ROWS, HID), jnp.float32),
        ],
    )
    def k(dst_hbm, z_hbm, o_hbm, out_hbm, didx, ones_v, acc):
        cid = lax.axis_index("c")
        sid = lax.axis_index("s")
        wid = cid * NS + sid
        pltpu.sync_copy(z_hbm, acc.at[pl.ds(sid * DRPS, DRPS), :])
        pltpu.sync_copy(dst_hbm.at[pl.ds(wid * K, K), :], didx)
        pltpu.sync_copy(o_hbm, ones_v)
        plsc.subcore_barrier()

        @pl.loop(0, K)
        def _(j):
            pltpu.sync_copy(ones_v, acc.at[didx.at[j]], add=True)

        plsc.subcore_barrier()
        pltpu.sync_copy(acc.at[pl.ds(sid * DRPS, DRPS), :],
                        out_hbm.at[cid, pl.ds(sid * DRPS, DRPS), :])

    return k(dst2d, z128, o128)


def _sc_propagate(g, src2d, dst2d, z128):
    @functools.partial(
        pl.kernel, mesh=_mesh,
        out_type=jax.ShapeDtypeStruct((NC, ACC_ROWS, HID), jnp.float32),
        scratch_types=[
            pltpu.VMEM((SW, CHUNK), jnp.int32),
            pltpu.VMEM((SW, CHUNK), jnp.int32),
            pltpu.VMEM((CHUNK, HID), jnp.float32),
            pltpu.VMEM((CHUNK, HID), jnp.float32),
            pltpu.VMEM_SHARED((ACC_ROWS, HID), jnp.float32),
            pltpu.SemaphoreType.DMA,
            pltpu.SemaphoreType.DMA,
            pltpu.SemaphoreType.DMA,
            pltpu.SemaphoreType.DMA,
        ],
    )
    def k(g_hbm, src_hbm, dst_hbm, z_hbm, out_hbm,
          sidx, didx, rows_a, rows_b, acc, sem_a, sem_b, sem_sa, sem_sb):
        cid = lax.axis_index("c")
        sid = lax.axis_index("s")
        wid = cid * NS + sid
        pltpu.sync_copy(z_hbm, acc.at[pl.ds(sid * RPS, RPS), :])
        plsc.subcore_barrier()

        def gstart(j, buf, sem):
            pltpu.make_async_copy(g_hbm.at[sidx.at[j]], buf, sem).start()

        def gwait(buf, sem):
            pltpu.make_async_copy(g_hbm.at[sidx.at[0]], buf, sem).wait()

        def sstart(j, buf, sem):
            pltpu.async_copy(buf, acc.at[didx.at[j]], sem, add=True)

        def swait(j, buf, sem):
            pltpu.make_async_copy(buf, acc.at[didx.at[0]], sem).wait()

        def sweep(cbase, n):
            pltpu.sync_copy(src_hbm.at[pl.ds(cbase, n), :],
                            sidx.at[pl.ds(0, n), :])
            pltpu.sync_copy(dst_hbm.at[pl.ds(cbase, n), :],
                            didx.at[pl.ds(0, n), :])
            gstart(0, rows_a, sem_a)
            gstart(1, rows_b, sem_b)

            @pl.loop(0, n - 4, step=2)
            def _(j):
                gwait(rows_a, sem_a)
                sstart(j, rows_a, sem_sa)
                gwait(rows_b, sem_b)
                sstart(j + 1, rows_b, sem_sb)
                swait(j, rows_a, sem_sa)
                gstart(j + 2, rows_a, sem_a)
                swait(j + 1, rows_b, sem_sb)
                gstart(j + 3, rows_b, sem_b)

            gwait(rows_a, sem_a)
            sstart(n - 4, rows_a, sem_sa)
            gwait(rows_b, sem_b)
            sstart(n - 3, rows_b, sem_sb)
            swait(n - 4, rows_a, sem_sa)
            gstart(n - 2, rows_a, sem_a)
            swait(n - 3, rows_b, sem_sb)
            gstart(n - 1, rows_b, sem_b)
            gwait(rows_a, sem_a)
            sstart(n - 2, rows_a, sem_sa)
            gwait(rows_b, sem_b)
            sstart(n - 1, rows_b, sem_sb)
            swait(n - 2, rows_a, sem_sa)
            swait(n - 1, rows_b, sem_sb)

        sweep(wid * K, SW)
        sweep(wid * K + SW, SW)

        plsc.subcore_barrier()
        pltpu.sync_copy(acc.at[pl.ds(sid * RPS, RPS), :],
                        out_hbm.at[cid, pl.ds(sid * RPS, RPS), :])

    return k(g, src2d, dst2d, z128)


def _tc_u1(x, batch2d, drone_feat, w1a, w1b):
    def body(x_ref, b_ref, df_ref, wa_ref, wb_ref, o_ref):
        dfw = jnp.dot(df_ref[...], wb_ref[...],
                      preferred_element_type=jnp.float32)
        gid = lax.broadcasted_iota(jnp.int32, (RBLK, N_GRAPHS), 1)
        oh = (b_ref[...] == gid).astype(jnp.float32)
        o_ref[...] = (
            jnp.dot(x_ref[...], wa_ref[...], preferred_element_type=jnp.float32)
            + jnp.dot(oh, dfw, preferred_element_type=jnp.float32))

    return pl.pallas_call(
        body,
        grid=(GRID,),
        in_specs=[
            pl.BlockSpec((RBLK, D_X), lambda i: (i, 0)),
            pl.BlockSpec((RBLK, 1), lambda i: (i, 0)),
            pl.BlockSpec((N_GRAPHS, 8), lambda i: (0, 0)),
            pl.BlockSpec((D_X, HID), lambda i: (0, 0)),
            pl.BlockSpec((8, HID), lambda i: (0, 0)),
        ],
        out_specs=pl.BlockSpec((RBLK, HID), lambda i: (i, 0)),
        out_shape=jax.ShapeDtypeStruct((N_NODES, HID), jnp.float32),
    )(x, batch2d, drone_feat, w1a, w1b)


def _dis_block(d0_ref, d1_ref):
    deg = 1.0 + d0_ref[...][:, 0:1] + d1_ref[...][:, 0:1]
    return lax.rsqrt(deg)


def _tc_scale(u, d0, d1):
    def body(u_ref, d0_ref, d1_ref, o_ref):
        o_ref[...] = u_ref[...] * _dis_block(d0_ref, d1_ref)

    return pl.pallas_call(
        body,
        grid=(GRID,),
        in_specs=[
            pl.BlockSpec((RBLK, HID), lambda i: (i, 0)),
            pl.BlockSpec((RBLK, HID), lambda i: (i, 0)),
            pl.BlockSpec((RBLK, HID), lambda i: (i, 0)),
        ],
        out_specs=pl.BlockSpec((RBLK, HID), lambda i: (i, 0)),
        out_shape=jax.ShapeDtypeStruct((N_NODES, HID), jnp.float32),
    )(u, d0, d1)


def _tc_mid(p0, p1, g, w_next, b_prev, d0, d1):
    def body(p0_ref, p1_ref, g_ref, w_ref, b_ref, d0_ref, d1_ref, o_ref):
        dis = _dis_block(d0_ref, d1_ref)
        h = jnp.maximum(
            dis * (p0_ref[...] + p1_ref[...] + g_ref[...]) + b_ref[...], 0.0)
        o_ref[...] = dis * jnp.dot(h, w_ref[...],
                                   preferred_element_type=jnp.float32)

    return pl.pallas_call(
        body,
        grid=(GRID,),
        in_specs=[
            pl.BlockSpec((RBLK, HID), lambda i: (i, 0)),
            pl.BlockSpec((RBLK, HID), lambda i: (i, 0)),
            pl.BlockSpec((RBLK, HID), lambda i: (i, 0)),
            pl.BlockSpec((HID, HID), lambda i: (0, 0)),
            pl.BlockSpec((1, HID), lambda i: (0, 0)),
            pl.BlockSpec((RBLK, HID), lambda i: (i, 0)),
            pl.BlockSpec((RBLK, HID), lambda i: (i, 0)),
        ],
        out_specs=pl.BlockSpec((RBLK, HID), lambda i: (i, 0)),
        out_shape=jax.ShapeDtypeStruct((N_NODES, HID), jnp.float32),
    )(p0, p1, g, w_next, b_prev, d0, d1)


def _tc_final(p0, p1, g, b, d0, d1):
    def body(p0_ref, p1_ref, g_ref, b_ref, d0_ref, d1_ref, o_ref):
        dis = _dis_block(d0_ref, d1_ref)
        o_ref[...] = dis * (p0_ref[...] + p1_ref[...] + g_ref[...]) + b_ref[...]

    return pl.pallas_call(
        body,
        grid=(GRID,),
        in_specs=[
            pl.BlockSpec((RBLK, HID), lambda i: (i, 0)),
            pl.BlockSpec((RBLK, HID), lambda i: (i, 0)),
            pl.BlockSpec((RBLK, HID), lambda i: (i, 0)),
            pl.BlockSpec((1, HID), lambda i: (0, 0)),
            pl.BlockSpec((RBLK, HID), lambda i: (i, 0)),
            pl.BlockSpec((RBLK, HID), lambda i: (i, 0)),
        ],
        out_specs=pl.BlockSpec((RBLK, HID), lambda i: (i, 0)),
        out_shape=jax.ShapeDtypeStruct((N_NODES, HID), jnp.float32),
    )(p0, p1, g, b, d0, d1)


def kernel(x, edge_index, drone_feat, batch, W1, b1, W2, b2, W3, b3):
    src = edge_index[0].astype(jnp.int32)
    dst = edge_index[1].astype(jnp.int32)
    pad = E_PAD - N_EDGES
    src2d = jnp.concatenate(
        [src, jnp.arange(pad, dtype=jnp.int32) % N_NODES]
    ).reshape(E_PAD // CHUNK, CHUNK)
    dst2d = jnp.concatenate(
        [dst, jnp.full((pad,), N_NODES, jnp.int32)]).reshape(E_PAD // CHUNK, CHUNK)
    z128 = jnp.zeros((RPS, HID), jnp.float32)
    zdeg = jnp.zeros((DRPS, HID), jnp.float32)
    odeg = jnp.ones((CHUNK, HID), jnp.float32)

    degp = _sc_degree(dst2d, zdeg, odeg)
    d0, d1 = degp[0], degp[1]

    u1 = _tc_u1(x, batch[:, None].astype(jnp.int32), drone_feat,
                W1[:D_X], W1[D_X:])
    g1 = _tc_scale(u1, d0, d1)

    p = _sc_propagate(g1, src2d, dst2d, z128)
    g2 = _tc_mid(p[0], p[1], g1, W2, b1[None, :], d0, d1)

    p = _sc_propagate(g2, src2d, dst2d, z128)
    g3 = _tc_mid(p[0], p[1], g2, W3, b2[None, :], d0, d1)

    p = _sc_propagate(g3, src2d, dst2d, z128)
    return _tc_final(p[0], p[1], g3, b3[None, :], d0, d1)


# speedup vs baseline: 1.2061x; 1.2061x over previous
import functools

import jax
import jax.numpy as jnp
from jax import lax
from jax.experimental import pallas as pl
from jax.experimental.pallas import tpu as pltpu
from jax.experimental.pallas import tpu_sc as plsc

N_NODES = 10000
N_EDGES = 320000
D_X = 120
IN_CH = 128
HID = 128
N_GRAPHS = 64

NC = 2
NS = 16
NW = NC * NS
CHUNK = 128
K = 80
E_PAD = NW * K * CHUNK
SW = 40
ACC_ROWS = 10112
RPS = ACC_ROWS // NS
DEG_ROWS = 10240
DRPS = DEG_ROWS // NS
RBLK = 2000
GRID = N_NODES // RBLK

_mesh = plsc.VectorSubcoreMesh(core_axis_name="c", subcore_axis_name="s")


def _sc_degree(dst2d, z128, o128):
    @functools.partial(
        pl.kernel, mesh=_mesh,
        out_type=jax.ShapeDtypeStruct((NC, DEG_ROWS, HID), jnp.float32),
        scratch_types=[
            pltpu.VMEM((K, CHUNK), jnp.int32),
            pltpu.VMEM((CHUNK, HID), jnp.float32),
            pltpu.VMEM_SHARED((DEG_ROWS, HID), jnp.float32),
        ],
    )
    def k(dst_hbm, z_hbm, o_hbm, out_hbm, didx, ones_v, acc):
        cid = lax.axis_index("c")
        sid = lax.axis_index("s")
        wid = cid * NS + sid
        pltpu.sync_copy(z_hbm, acc.at[pl.ds(sid * DRPS, DRPS), :])
        pltpu.sync_copy(dst_hbm.at[pl.ds(wid * K, K), :], didx)
        pltpu.sync_copy(o_hbm, ones_v)
        plsc.subcore_barrier()

        @pl.loop(0, K)
        def _(j):
            pltpu.sync_copy(ones_v, acc.at[didx.at[j]], add=True)

        plsc.subcore_barrier()
        pltpu.sync_copy(acc.at[pl.ds(sid * DRPS, DRPS), :],
                        out_hbm.at[cid, pl.ds(sid * DRPS, DRPS), :])

    return k(dst2d, z128, o128)


def _sc_propagate(g, src2d, dst2d, z128):
    @functools.partial(
        pl.kernel, mesh=_mesh,
        out_type=jax.ShapeDtypeStruct((NC, ACC_ROWS, HID), jnp.float32),
        scratch_types=[
            pltpu.VMEM((SW, CHUNK), jnp.int32),
            pltpu.VMEM((SW, CHUNK), jnp.int32),
            pltpu.VMEM((CHUNK, HID), jnp.float32),
            pltpu.VMEM((CHUNK, HID), jnp.float32),
            pltpu.VMEM_SHARED((ACC_ROWS, HID), jnp.float32),
            pltpu.SemaphoreType.DMA,
            pltpu.SemaphoreType.DMA,
        ],
    )
    def k(g_hbm, src_hbm, dst_hbm, z_hbm, out_hbm,
          sidx, didx, rows_a, rows_b, acc, sem_a, sem_b):
        cid = lax.axis_index("c")
        sid = lax.axis_index("s")
        wid = cid * NS + sid
        pltpu.sync_copy(z_hbm, acc.at[pl.ds(sid * RPS, RPS), :])
        plsc.subcore_barrier()

        def gstart(j, buf, sem):
            pltpu.make_async_copy(g_hbm.at[sidx.at[j]], buf, sem).start()

        def gwait(buf, sem):
            pltpu.make_async_copy(g_hbm.at[sidx.at[0]], buf, sem).wait()

        def scat(j, buf):
            pltpu.sync_copy(buf, acc.at[didx.at[j]], add=True)

        def sweep(cbase, n):
            pltpu.sync_copy(src_hbm.at[pl.ds(cbase, n), :],
                            sidx.at[pl.ds(0, n), :])
            pltpu.sync_copy(dst_hbm.at[pl.ds(cbase, n), :],
                            didx.at[pl.ds(0, n), :])
            gstart(0, rows_a, sem_a)

            @pl.loop(0, n - 2, step=2)
            def _(j):
                gstart(j + 1, rows_b, sem_b)
                gwait(rows_a, sem_a)
                scat(j, rows_a)
                gstart(j + 2, rows_a, sem_a)
                gwait(rows_b, sem_b)
                scat(j + 1, rows_b)

            gstart(n - 1, rows_b, sem_b)
            gwait(rows_a, sem_a)
            scat(n - 2, rows_a)
            gwait(rows_b, sem_b)
            scat(n - 1, rows_b)

        sweep(wid * K, SW)
        sweep(wid * K + SW, SW)

        plsc.subcore_barrier()
        pltpu.sync_copy(acc.at[pl.ds(sid * RPS, RPS), :],
                        out_hbm.at[cid, pl.ds(sid * RPS, RPS), :])

    return k(g, src2d, dst2d, z128)


def _tc_u1(x, batch2d, drone_feat, w1a, w1b):
    def body(x_ref, b_ref, df_ref, wa_ref, wb_ref, o_ref):
        dfw = jnp.dot(df_ref[...], wb_ref[...],
                      preferred_element_type=jnp.float32)
        gid = lax.broadcasted_iota(jnp.int32, (RBLK, N_GRAPHS), 1)
        oh = (b_ref[...] == gid).astype(jnp.float32)
        o_ref[...] = (
            jnp.dot(x_ref[...], wa_ref[...], preferred_element_type=jnp.float32)
            + jnp.dot(oh, dfw, preferred_element_type=jnp.float32))

    return pl.pallas_call(
        body,
        grid=(GRID,),
        in_specs=[
            pl.BlockSpec((RBLK, D_X), lambda i: (i, 0)),
            pl.BlockSpec((RBLK, 1), lambda i: (i, 0)),
            pl.BlockSpec((N_GRAPHS, 8), lambda i: (0, 0)),
            pl.BlockSpec((D_X, HID), lambda i: (0, 0)),
            pl.BlockSpec((8, HID), lambda i: (0, 0)),
        ],
        out_specs=pl.BlockSpec((RBLK, HID), lambda i: (i, 0)),
        out_shape=jax.ShapeDtypeStruct((N_NODES, HID), jnp.float32),
    )(x, batch2d, drone_feat, w1a, w1b)


def _dis_block(d0_ref, d1_ref):
    deg = 1.0 + d0_ref[...][:, 0:1] + d1_ref[...][:, 0:1]
    return lax.rsqrt(deg)


def _tc_scale(u, d0, d1):
    def body(u_ref, d0_ref, d1_ref, o_ref):
        o_ref[...] = u_ref[...] * _dis_block(d0_ref, d1_ref)

    return pl.pallas_call(
        body,
        grid=(GRID,),
        in_specs=[
            pl.BlockSpec((RBLK, HID), lambda i: (i, 0)),
            pl.BlockSpec((RBLK, HID), lambda i: (i, 0)),
            pl.BlockSpec((RBLK, HID), lambda i: (i, 0)),
        ],
        out_specs=pl.BlockSpec((RBLK, HID), lambda i: (i, 0)),
        out_shape=jax.ShapeDtypeStruct((N_NODES, HID), jnp.float32),
    )(u, d0, d1)


def _tc_mid(p0, p1, g, w_next, b_prev, d0, d1):
    def body(p0_ref, p1_ref, g_ref, w_ref, b_ref, d0_ref, d1_ref, o_ref):
        dis = _dis_block(d0_ref, d1_ref)
        h = jnp.maximum(
            dis * (p0_ref[...] + p1_ref[...] + g_ref[...]) + b_ref[...], 0.0)
        o_ref[...] = dis * jnp.dot(h, w_ref[...],
                                   preferred_element_type=jnp.float32)

    return pl.pallas_call(
        body,
        grid=(GRID,),
        in_specs=[
            pl.BlockSpec((RBLK, HID), lambda i: (i, 0)),
            pl.BlockSpec((RBLK, HID), lambda i: (i, 0)),
            pl.BlockSpec((RBLK, HID), lambda i: (i, 0)),
            pl.BlockSpec((HID, HID), lambda i: (0, 0)),
            pl.BlockSpec((1, HID), lambda i: (0, 0)),
            pl.BlockSpec((RBLK, HID), lambda i: (i, 0)),
            pl.BlockSpec((RBLK, HID), lambda i: (i, 0)),
        ],
        out_specs=pl.BlockSpec((RBLK, HID), lambda i: (i, 0)),
        out_shape=jax.ShapeDtypeStruct((N_NODES, HID), jnp.float32),
    )(p0, p1, g, w_next, b_prev, d0, d1)


def _tc_final(p0, p1, g, b, d0, d1):
    def body(p0_ref, p1_ref, g_ref, b_ref, d0_ref, d1_ref, o_ref):
        dis = _dis_block(d0_ref, d1_ref)
        o_ref[...] = dis * (p0_ref[...] + p1_ref[...] + g_ref[...]) + b_ref[...]

    return pl.pallas_call(
        body,
        grid=(GRID,),
        in_specs=[
            pl.BlockSpec((RBLK, HID), lambda i: (i, 0)),
            pl.BlockSpec((RBLK, HID), lambda i: (i, 0)),
            pl.BlockSpec((RBLK, HID), lambda i: (i, 0)),
            pl.BlockSpec((1, HID), lambda i: (0, 0)),
            pl.BlockSpec((RBLK, HID), lambda i: (i, 0)),
            pl.BlockSpec((RBLK, HID), lambda i: (i, 0)),
        ],
        out_specs=pl.BlockSpec((RBLK, HID), lambda i: (i, 0)),
        out_shape=jax.ShapeDtypeStruct((N_NODES, HID), jnp.float32),
    )(p0, p1, g, b, d0, d1)


def kernel(x, edge_index, drone_feat, batch, W1, b1, W2, b2, W3, b3):
    src = edge_index[0].astype(jnp.int32)
    dst = edge_index[1].astype(jnp.int32)
    pad = E_PAD - N_EDGES
    src2d = jnp.concatenate(
        [src, jnp.arange(pad, dtype=jnp.int32) % N_NODES]
    ).reshape(E_PAD // CHUNK, CHUNK)
    dst2d = jnp.concatenate(
        [dst, jnp.full((pad,), N_NODES, jnp.int32)]).reshape(E_PAD // CHUNK, CHUNK)
    z128 = jnp.zeros((RPS, HID), jnp.float32)
    zdeg = jnp.zeros((DRPS, HID), jnp.float32)
    odeg = jnp.ones((CHUNK, HID), jnp.float32)

    degp = _sc_degree(dst2d, zdeg, odeg)
    d0, d1 = degp[0], degp[1]

    u1 = _tc_u1(x, batch[:, None].astype(jnp.int32), drone_feat,
                W1[:D_X], W1[D_X:])
    g1 = _tc_scale(u1, d0, d1)

    p = _sc_propagate(g1, src2d, dst2d, z128)
    g2 = _tc_mid(p[0], p[1], g1, W2, b1[None, :], d0, d1)

    p = _sc_propagate(g2, src2d, dst2d, z128)
    g3 = _tc_mid(p[0], p[1], g2, W3, b2[None, :], d0, d1)

    p = _sc_propagate(g3, src2d, dst2d, z128)
    return _tc_final(p[0], p[1], g3, b3[None, :], d0, d1)
